# baseline (speedup 1.0000x reference)
import functools

import numpy as np
import jax
import jax.numpy as jnp
from jax import lax
from jax.experimental import pallas as pl
from jax.experimental.pallas import tpu as pltpu
from jax.experimental.pallas import tpu_sc as plsc

B = 4096
V = 100000
C = 64
T = 50
P = 3 * T
D = P * C

NC, NS = 2, 16
NW = NC * NS
SPW = B // NW
SPC = 2
NCHUNK = SPW // SPC
ROWS = SPC * T
IDX_ROWS = B // SPC


def _pos_encoding() -> np.ndarray:
    half = C // 2
    positions = np.arange(P)[:, np.newaxis]
    dims = np.arange(half)[np.newaxis, :] / half
    rates = 1.0 / 10000 ** dims
    rads = positions * rates
    return np.concatenate([np.sin(rads), np.cos(rads)], axis=-1).astype(np.float32)


_POS = _pos_encoding()
_POS_FLAT = _POS.reshape(1, D)
_POS_I = np.tile(_POS[0::3], (SPC, 1))


def _tc_body(et_ref, w_ref, pos_ref, out_ref):
    out_ref[...] = (
        jnp.dot(et_ref[...], w_ref[...], preferred_element_type=jnp.float32)
        + pos_ref[...]
    )


def _tc_call(et, w_cat, posf, bb=256):
    return pl.pallas_call(
        _tc_body,
        grid=(B // bb,),
        in_specs=[
            pl.BlockSpec((bb, 2 * T), lambda i: (i, 0)),
            pl.BlockSpec((2 * T, D), lambda i: (0, 0)),
            pl.BlockSpec((1, D), lambda i: (0, 0)),
        ],
        out_specs=pl.BlockSpec((bb, D), lambda i: (i, 0)),
        out_shape=jax.ShapeDtypeStruct((B, D), jnp.float32),
    )(et, w_cat, posf)


def _sc_body(out_hbm, table_hbm, idx_hbm, pos_hbm,
             idx_v, pos_v, buf0, buf1, sem0, sem1):
    c = lax.axis_index("c")
    s = lax.axis_index("s")
    wid = s * NC + c
    row0 = wid * NCHUNK

    pltpu.sync_copy(idx_hbm.at[pl.ds(row0, NCHUNK)], idx_v)
    pltpu.sync_copy(pos_hbm, pos_v)

    bufs = (buf0, buf1)
    sems = (sem0, sem1)

    pltpu.async_copy(table_hbm.at[idx_v.at[0]], buf0, sem0)
    pltpu.async_copy(table_hbm.at[idx_v.at[1]], buf1, sem1)

    def step(i2, _):
        for b in range(2):
            j = i2 * 2 + b
            buf = bufs[b]
            pltpu.make_async_copy(
                table_hbm.at[idx_v.at[j]], buf, sems[b]).wait()
            for r in range(ROWS):
                for cc in range(4):
                    sl = pl.ds(cc * 16, 16)
                    buf[r, sl] += pos_v[r, sl]
            g0 = (row0 + j) * SPC
            for sbl in range(SPC):
                pltpu.sync_copy(
                    buf.at[pl.ds(sbl * T, T)],
                    out_hbm.at[g0 + sbl, :, 0, :])
            @pl.when(j + 2 < NCHUNK)
            def _():
                pltpu.async_copy(
                    table_hbm.at[idx_v.at[j + 2]], buf, sems[b])
        return 0

    lax.fori_loop(0, NCHUNK // 2, step, 0)


@functools.cache
def _sc_fill():
    return pl.kernel(
        _sc_body,
        out_type=(),
        mesh=plsc.VectorSubcoreMesh(
            core_axis_name="c", subcore_axis_name="s",
            num_cores=NC, num_subcores=NS),
        scratch_types=[
            pltpu.VMEM((NCHUNK, ROWS), jnp.int32),
            pltpu.VMEM((ROWS, C), jnp.float32),
            pltpu.VMEM((ROWS, C), jnp.float32),
            pltpu.VMEM((ROWS, C), jnp.float32),
            pltpu.SemaphoreType.DMA,
            pltpu.SemaphoreType.DMA,
        ],
        compiler_params=pltpu.CompilerParams(use_tc_tiling_on_sc=False),
    )


def kernel(x, emb_table, W_e, W_t):
    x3 = x.reshape(B, T, 3)
    et = jnp.concatenate([x3[:, :, 1], x3[:, :, 2]], axis=1)
    idx = x3[:, :, 0].astype(jnp.int32).reshape(IDX_ROWS, ROWS)

    we3 = W_e.reshape(T, T, 1, C)
    wt3 = W_t.reshape(T, T, 1, C)
    z = jnp.zeros((T, T, 1, C), jnp.float32)
    top = jnp.concatenate([z, we3, z], axis=2).reshape(T, D)
    bot = jnp.concatenate([z, z, wt3], axis=2).reshape(T, D)
    w_cat = jnp.concatenate([top, bot], axis=0)

    posf = jnp.asarray(_POS_FLAT)
    out_flat = _tc_call(et, w_cat, posf)

    out_ref = jax.new_ref(out_flat.reshape(B, T, 3, C))
    _sc_fill()(out_ref, emb_table, idx, jnp.asarray(_POS_I))
    return out_ref[...].reshape(B, P, C)


# speedup vs baseline: 3.8045x; 3.8045x over previous
import functools

import numpy as np
import jax
import jax.numpy as jnp
from jax import lax
from jax.experimental import pallas as pl
from jax.experimental.pallas import tpu as pltpu
from jax.experimental.pallas import tpu_sc as plsc

B = 4096
V = 100000
C = 64
T = 50
P = 3 * T
D = P * C

NC, NS = 2, 16
NW = NC * NS
SPW = B // NW
SPC = 2
NCHUNK = SPW // SPC
ROWS = SPC * T
IDX_ROWS = B // SPC


def _pos_encoding() -> np.ndarray:
    half = C // 2
    positions = np.arange(P)[:, np.newaxis]
    dims = np.arange(half)[np.newaxis, :] / half
    rates = 1.0 / 10000 ** dims
    rads = positions * rates
    return np.concatenate([np.sin(rads), np.cos(rads)], axis=-1).astype(np.float32)


_POS = _pos_encoding()
_POS_FLAT = _POS.reshape(1, D)
_POS_I = _POS[0::3].reshape(1, T * C)


def _tc_body(et_ref, w_ref, posf_ref, posi_ref, g_ref, out_ref):
    acc = (
        jnp.dot(et_ref[...], w_ref[...], preferred_element_type=jnp.float32)
        + posf_ref[...]
    )
    out_ref[...] = acc
    gp = g_ref[...] + posi_ref[...]
    for k in range(T):
        out_ref[:, 3 * k * C:(3 * k + 1) * C] = gp[:, k * C:(k + 1) * C]


def _tc_call(et, w_cat, posf, posi, g, bb=256):
    return pl.pallas_call(
        _tc_body,
        grid=(B // bb,),
        in_specs=[
            pl.BlockSpec((bb, 2 * T), lambda i: (i, 0)),
            pl.BlockSpec((2 * T, D), lambda i: (0, 0)),
            pl.BlockSpec((1, D), lambda i: (0, 0)),
            pl.BlockSpec((1, T * C), lambda i: (0, 0)),
            pl.BlockSpec((bb, T * C), lambda i: (i, 0)),
        ],
        out_specs=pl.BlockSpec((bb, D), lambda i: (i, 0)),
        out_shape=jax.ShapeDtypeStruct((B, D), jnp.float32),
    )(et, w_cat, posf, posi, g)


def _sc_body(table_hbm, idx_hbm, g_hbm, idx_v, buf0, buf1, sem0, sem1, osem):
    c = lax.axis_index("c")
    s = lax.axis_index("s")
    wid = s * NC + c
    row0 = wid * NCHUNK

    pltpu.sync_copy(idx_hbm.at[pl.ds(row0, NCHUNK)], idx_v)

    bufs = (buf0, buf1)
    sems = (sem0, sem1)

    pltpu.async_copy(table_hbm.at[idx_v.at[0]], buf0, sem0)
    pltpu.async_copy(table_hbm.at[idx_v.at[1]], buf1, sem1)

    def step(i2, _):
        for b in range(2):
            j = i2 * 2 + b
            buf = bufs[b]
            pltpu.make_async_copy(
                table_hbm.at[idx_v.at[j]], buf, sems[b]).wait()
            pltpu.async_copy(buf, g_hbm.at[row0 + j], osem)
            pltpu.make_async_copy(buf, g_hbm.at[row0 + j], osem).wait()
            @pl.when(j + 2 < NCHUNK)
            def _():
                pltpu.async_copy(
                    table_hbm.at[idx_v.at[j + 2]], buf, sems[b])
        return 0

    lax.fori_loop(0, NCHUNK // 2, step, 0)


@functools.cache
def _sc_gather():
    return pl.kernel(
        _sc_body,
        out_type=jax.ShapeDtypeStruct((IDX_ROWS, ROWS, C), jnp.float32),
        mesh=plsc.VectorSubcoreMesh(
            core_axis_name="c", subcore_axis_name="s",
            num_cores=NC, num_subcores=NS),
        scratch_types=[
            pltpu.VMEM((NCHUNK, ROWS), jnp.int32),
            pltpu.VMEM((ROWS, C), jnp.float32),
            pltpu.VMEM((ROWS, C), jnp.float32),
            pltpu.SemaphoreType.DMA,
            pltpu.SemaphoreType.DMA,
            pltpu.SemaphoreType.DMA,
        ],
        compiler_params=pltpu.CompilerParams(use_tc_tiling_on_sc=False),
    )


def kernel(x, emb_table, W_e, W_t):
    x3 = x.reshape(B, T, 3)
    et = jnp.concatenate([x3[:, :, 1], x3[:, :, 2]], axis=1)
    idx = x3[:, :, 0].astype(jnp.int32).reshape(IDX_ROWS, ROWS)

    g = _sc_gather()(emb_table, idx)

    we3 = W_e.reshape(T, T, 1, C)
    wt3 = W_t.reshape(T, T, 1, C)
    z = jnp.zeros((T, T, 1, C), jnp.float32)
    top = jnp.concatenate([z, we3, z], axis=2).reshape(T, D)
    bot = jnp.concatenate([z, z, wt3], axis=2).reshape(T, D)
    w_cat = jnp.concatenate([top, bot], axis=0)

    out_flat = _tc_call(
        et, w_cat, jnp.asarray(_POS_FLAT), jnp.asarray(_POS_I),
        g.reshape(B, T * C))
    return out_flat.reshape(B, P, C)


# speedup vs baseline: 3.8258x; 1.0056x over previous
import functools

import numpy as np
import jax
import jax.numpy as jnp
from jax import lax
from jax.experimental import pallas as pl
from jax.experimental.pallas import tpu as pltpu
from jax.experimental.pallas import tpu_sc as plsc

B = 4096
V = 100000
C = 64
T = 50
P = 3 * T
D = P * C

NC, NS = 2, 16
NW = NC * NS
SPW = B // NW
SPC = 2
NCHUNK = SPW // SPC
ROWS = SPC * T
IDX_ROWS = B // SPC


def _pos_encoding() -> np.ndarray:
    half = C // 2
    positions = np.arange(P)[:, np.newaxis]
    dims = np.arange(half)[np.newaxis, :] / half
    rates = 1.0 / 10000 ** dims
    rads = positions * rates
    return np.concatenate([np.sin(rads), np.cos(rads)], axis=-1).astype(np.float32)


_POS = _pos_encoding()
_POS_FLAT = _POS.reshape(1, D)
_POS_I = _POS[0::3].reshape(1, T * C)


def _tc_body(et_ref, w_ref, posf_ref, posi_ref, g_ref, out_ref):
    acc = (
        jnp.dot(et_ref[...], w_ref[...], preferred_element_type=jnp.float32)
        + posf_ref[...]
    )
    out_ref[...] = acc
    gp = g_ref[...] + posi_ref[...]
    for k in range(T):
        out_ref[:, 3 * k * C:(3 * k + 1) * C] = gp[:, k * C:(k + 1) * C]


def _tc_call(et, w_cat, posf, posi, g, bb=512):
    return pl.pallas_call(
        _tc_body,
        grid=(B // bb,),
        in_specs=[
            pl.BlockSpec((bb, 2 * T), lambda i: (i, 0)),
            pl.BlockSpec((2 * T, D), lambda i: (0, 0)),
            pl.BlockSpec((1, D), lambda i: (0, 0)),
            pl.BlockSpec((1, T * C), lambda i: (0, 0)),
            pl.BlockSpec((bb, T * C), lambda i: (i, 0)),
        ],
        out_specs=pl.BlockSpec((bb, D), lambda i: (i, 0)),
        out_shape=jax.ShapeDtypeStruct((B, D), jnp.float32),
    )(et, w_cat, posf, posi, g)


def _sc_body(table_hbm, idx_hbm, g_hbm, idx_v, buf0, buf1, sem0, sem1, osem):
    c = lax.axis_index("c")
    s = lax.axis_index("s")
    wid = s * NC + c
    row0 = wid * NCHUNK

    pltpu.sync_copy(idx_hbm.at[pl.ds(row0, NCHUNK)], idx_v)

    bufs = (buf0, buf1)
    sems = (sem0, sem1)

    pltpu.async_copy(table_hbm.at[idx_v.at[0]], buf0, sem0)
    pltpu.async_copy(table_hbm.at[idx_v.at[1]], buf1, sem1)

    def step(i2, _):
        for b in range(2):
            j = i2 * 2 + b
            buf = bufs[b]
            pltpu.make_async_copy(
                table_hbm.at[idx_v.at[j]], buf, sems[b]).wait()
            pltpu.async_copy(buf, g_hbm.at[row0 + j], osem)
            pltpu.make_async_copy(buf, g_hbm.at[row0 + j], osem).wait()
            @pl.when(j + 2 < NCHUNK)
            def _():
                pltpu.async_copy(
                    table_hbm.at[idx_v.at[j + 2]], buf, sems[b])
        return 0

    lax.fori_loop(0, NCHUNK // 2, step, 0)


@functools.cache
def _sc_gather():
    return pl.kernel(
        _sc_body,
        out_type=jax.ShapeDtypeStruct((IDX_ROWS, ROWS, C), jnp.float32),
        mesh=plsc.VectorSubcoreMesh(
            core_axis_name="c", subcore_axis_name="s",
            num_cores=NC, num_subcores=NS),
        scratch_types=[
            pltpu.VMEM((NCHUNK, ROWS), jnp.int32),
            pltpu.VMEM((ROWS, C), jnp.float32),
            pltpu.VMEM((ROWS, C), jnp.float32),
            pltpu.SemaphoreType.DMA,
            pltpu.SemaphoreType.DMA,
            pltpu.SemaphoreType.DMA,
        ],
        compiler_params=pltpu.CompilerParams(use_tc_tiling_on_sc=False),
    )


def kernel(x, emb_table, W_e, W_t):
    x3 = x.reshape(B, T, 3)
    et = jnp.concatenate([x3[:, :, 1], x3[:, :, 2]], axis=1)
    idx = x3[:, :, 0].astype(jnp.int32).reshape(IDX_ROWS, ROWS)

    g = _sc_gather()(emb_table, idx)

    we3 = W_e.reshape(T, T, 1, C)
    wt3 = W_t.reshape(T, T, 1, C)
    z = jnp.zeros((T, T, 1, C), jnp.float32)
    top = jnp.concatenate([z, we3, z], axis=2).reshape(T, D)
    bot = jnp.concatenate([z, z, wt3], axis=2).reshape(T, D)
    w_cat = jnp.concatenate([top, bot], axis=0)

    out_flat = _tc_call(
        et, w_cat, jnp.asarray(_POS_FLAT), jnp.asarray(_POS_I),
        g.reshape(B, T * C))
    return out_flat.reshape(B, P, C)


# speedup vs baseline: 3.8829x; 1.0149x over previous
import functools

import numpy as np
import jax
import jax.numpy as jnp
from jax import lax
from jax.experimental import pallas as pl
from jax.experimental.pallas import tpu as pltpu
from jax.experimental.pallas import tpu_sc as plsc

B = 4096
V = 100000
C = 64
T = 50
P = 3 * T
D = P * C

NC, NS = 2, 16
NW = NC * NS
SPW = B // NW
SPC = 2
NCHUNK = SPW // SPC
ROWS = SPC * T
IDX_ROWS = B // SPC


def _pos_encoding() -> np.ndarray:
    half = C // 2
    positions = np.arange(P)[:, np.newaxis]
    dims = np.arange(half)[np.newaxis, :] / half
    rates = 1.0 / 10000 ** dims
    rads = positions * rates
    return np.concatenate([np.sin(rads), np.cos(rads)], axis=-1).astype(np.float32)


_POS = _pos_encoding()
_POS_FLAT = _POS.reshape(1, D)
_POS_I = _POS[0::3].reshape(1, T * C)


def _tc_body(et_ref, w_ref, posf_ref, posi_ref, g_ref, out_ref):
    acc = (
        jnp.dot(et_ref[...], w_ref[...], preferred_element_type=jnp.float32)
        + posf_ref[...]
    )
    out_ref[...] = acc
    gp = g_ref[...] + posi_ref[...]
    for k in range(T):
        out_ref[:, 3 * k * C:(3 * k + 1) * C] = gp[:, k * C:(k + 1) * C]


def _tc_call(et, w_cat, posf, posi, g, bb=512):
    return pl.pallas_call(
        _tc_body,
        grid=(B // bb,),
        in_specs=[
            pl.BlockSpec((bb, 2 * T), lambda i: (i, 0)),
            pl.BlockSpec((2 * T, D), lambda i: (0, 0)),
            pl.BlockSpec((1, D), lambda i: (0, 0)),
            pl.BlockSpec((1, T * C), lambda i: (0, 0)),
            pl.BlockSpec((bb, T * C), lambda i: (i, 0)),
        ],
        out_specs=pl.BlockSpec((bb, D), lambda i: (i, 0)),
        out_shape=jax.ShapeDtypeStruct((B, D), jnp.float32),
    )(et, w_cat, posf, posi, g)


_NBUF = 4


def _sc_body(table_hbm, idx_hbm, g_hbm, idx_v,
             buf0, buf1, buf2, buf3,
             gs0, gs1, gs2, gs3, os0, os1, os2, os3):
    c = lax.axis_index("c")
    s = lax.axis_index("s")
    wid = s * NC + c
    row0 = wid * NCHUNK

    pltpu.sync_copy(idx_hbm.at[pl.ds(row0, NCHUNK)], idx_v)

    bufs = (buf0, buf1, buf2, buf3)
    gsems = (gs0, gs1, gs2, gs3)
    osems = (os0, os1, os2, os3)

    for j in range(3):
        pltpu.async_copy(table_hbm.at[idx_v.at[j]], bufs[j], gsems[j])

    for j in range(NCHUNK):
        b = j % _NBUF
        pltpu.make_async_copy(
            table_hbm.at[idx_v.at[j]], bufs[b], gsems[b]).wait()
        pltpu.async_copy(bufs[b], g_hbm.at[row0 + j], osems[b])
        nj = j + 3
        if nj < NCHUNK:
            bn = nj % _NBUF
            if nj >= _NBUF:
                pltpu.make_async_copy(
                    bufs[bn], g_hbm.at[row0 + nj - _NBUF], osems[bn]).wait()
            pltpu.async_copy(table_hbm.at[idx_v.at[nj]], bufs[bn], gsems[bn])

    for j in range(NCHUNK - _NBUF, NCHUNK):
        if j >= 0:
            b = j % _NBUF
            pltpu.make_async_copy(
                bufs[b], g_hbm.at[row0 + j], osems[b]).wait()


@functools.cache
def _sc_gather():
    return pl.kernel(
        _sc_body,
        out_type=jax.ShapeDtypeStruct((IDX_ROWS, ROWS, C), jnp.float32),
        mesh=plsc.VectorSubcoreMesh(
            core_axis_name="c", subcore_axis_name="s",
            num_cores=NC, num_subcores=NS),
        scratch_types=[
            pltpu.VMEM((NCHUNK, ROWS), jnp.int32),
            pltpu.VMEM((ROWS, C), jnp.float32),
            pltpu.VMEM((ROWS, C), jnp.float32),
            pltpu.VMEM((ROWS, C), jnp.float32),
            pltpu.VMEM((ROWS, C), jnp.float32),
            pltpu.SemaphoreType.DMA,
            pltpu.SemaphoreType.DMA,
            pltpu.SemaphoreType.DMA,
            pltpu.SemaphoreType.DMA,
            pltpu.SemaphoreType.DMA,
            pltpu.SemaphoreType.DMA,
            pltpu.SemaphoreType.DMA,
            pltpu.SemaphoreType.DMA,
        ],
        compiler_params=pltpu.CompilerParams(use_tc_tiling_on_sc=False),
    )


def kernel(x, emb_table, W_e, W_t):
    x3 = x.reshape(B, T, 3)
    et = jnp.concatenate([x3[:, :, 1], x3[:, :, 2]], axis=1)
    idx = x3[:, :, 0].astype(jnp.int32).reshape(IDX_ROWS, ROWS)

    g = _sc_gather()(emb_table, idx)

    we3 = W_e.reshape(T, T, 1, C)
    wt3 = W_t.reshape(T, T, 1, C)
    z = jnp.zeros((T, T, 1, C), jnp.float32)
    top = jnp.concatenate([z, we3, z], axis=2).reshape(T, D)
    bot = jnp.concatenate([z, z, wt3], axis=2).reshape(T, D)
    w_cat = jnp.concatenate([top, bot], axis=0)

    out_flat = _tc_call(
        et, w_cat, jnp.asarray(_POS_FLAT), jnp.asarray(_POS_I),
        g.reshape(B, T * C))
    return out_flat.reshape(B, P, C)
